# baseline (speedup 1.0000x reference)
import functools

import jax
import jax.numpy as jnp
from jax import lax
from jax.experimental import pallas as pl
from jax.experimental.pallas import tpu as pltpu
from jax.experimental.pallas import tpu_sc as plsc

V = 1000000
D = 64
R = 16
N = 1024 * 200

NC = 2
NS = 16
NW = NC * NS
B_PER_W = N // NW
CHUNK = 640
NCHUNKS = B_PER_W // CHUNK


def _worker_id():
    return lax.axis_index("s") * NC + lax.axis_index("c")


@functools.cache
def _sc_gather():
    mesh = plsc.VectorSubcoreMesh(core_axis_name="c", subcore_axis_name="s")

    @functools.partial(
        pl.kernel,
        out_type=(
            jax.ShapeDtypeStruct((2 * N, 32), jnp.float32),
            jax.ShapeDtypeStruct((N, R), jnp.float32),
        ),
        mesh=mesh,
        compiler_params=pltpu.CompilerParams(use_tc_tiling_on_sc=False),
        scratch_types=[
            pltpu.VMEM((2 * B_PER_W,), jnp.int32),
            pltpu.VMEM((B_PER_W,), jnp.int32),
            pltpu.VMEM((2 * CHUNK, 32), jnp.float32),
            pltpu.VMEM((2 * CHUNK, 32), jnp.float32),
            pltpu.VMEM((CHUNK, R), jnp.float32),
            pltpu.VMEM((CHUNK, R), jnp.float32),
            pltpu.SemaphoreType.DMA,
            pltpu.SemaphoreType.DMA,
        ],
    )
    def gather(idx2_hbm, idx_hbm, table2_hbm, at_hbm, out_b_hbm, out_a_hbm,
               idx2_v, idx_v, b0, b1, a0, a1, sem0, sem1):
        wid = _worker_id()
        base = wid * B_PER_W
        pltpu.sync_copy(idx2_hbm.at[pl.ds(2 * base, 2 * B_PER_W)], idx2_v)
        pltpu.sync_copy(idx_hbm.at[pl.ds(base, B_PER_W)], idx_v)
        bbufs = (b0, b1)
        abufs = (a0, a1)
        sems = (sem0, sem1)
        cps = [None, None]

        def start(k):
            j = k % 2
            cb = pltpu.async_copy(
                table2_hbm.at[idx2_v.at[pl.ds(2 * k * CHUNK, 2 * CHUNK)]],
                bbufs[j], sems[j])
            ca = pltpu.async_copy(
                at_hbm.at[idx_v.at[pl.ds(k * CHUNK, CHUNK)]],
                abufs[j], sems[j])
            cps[j] = (cb, ca)

        start(0)
        for k in range(NCHUNKS):
            if k + 1 < NCHUNKS:
                start(k + 1)
            j = k % 2
            cps[j][0].wait()
            cps[j][1].wait()
            pltpu.sync_copy(
                bbufs[j], out_b_hbm.at[pl.ds(2 * (base + k * CHUNK), 2 * CHUNK)])
            pltpu.sync_copy(
                abufs[j], out_a_hbm.at[pl.ds(base + k * CHUNK, CHUNK)])

    return gather


_VB = 2048


def _transpose_body(a_ref, out_ref):
    out_ref[...] = a_ref[...].T


_transpose = pl.pallas_call(
    _transpose_body,
    grid=(pl.cdiv(V, _VB),),
    in_specs=[pl.BlockSpec((R, _VB), lambda i: (0, i))],
    out_specs=pl.BlockSpec((_VB, R), lambda i: (i, 0)),
    out_shape=jax.ShapeDtypeStruct((V, R), jnp.float32),
)

_BN = 2048


def _fuse_body(ar_ref, rows_ref, b_ref, out_ref):
    out_ref[...] = rows_ref[...] + jnp.dot(
        ar_ref[...], b_ref[...].T, preferred_element_type=jnp.float32
    )


_fuse = pl.pallas_call(
    _fuse_body,
    grid=(N // _BN,),
    in_specs=[
        pl.BlockSpec((_BN, R), lambda i: (i, 0)),
        pl.BlockSpec((_BN, D), lambda i: (i, 0)),
        pl.BlockSpec((D, R), lambda i: (0, 0)),
    ],
    out_specs=pl.BlockSpec((_BN, D), lambda i: (i, 0)),
    out_shape=jax.ShapeDtypeStruct((N, D), jnp.float32),
)


def kernel(x, base_weight, lora_A, lora_B):
    Bsz, Ssz = x.shape
    idx = x.reshape(-1)
    idx2 = (2 * idx[:, None] + jnp.arange(2, dtype=jnp.int32)[None, :]).reshape(-1)
    table2 = base_weight.reshape(2 * V, 32)
    at = _transpose(lora_A)
    rows2, ar = _sc_gather()(idx2, idx, table2, at)
    out = _fuse(ar, rows2.reshape(N, D), lora_B)
    return out.reshape(Bsz, Ssz, D)


# speedup vs baseline: 1.0527x; 1.0527x over previous
import functools

import jax
import jax.numpy as jnp
from jax import lax
from jax.experimental import pallas as pl
from jax.experimental.pallas import tpu as pltpu
from jax.experimental.pallas import tpu_sc as plsc

V = 1000000
D = 64
R = 16
N = 1024 * 200

NC = 2
NS = 16
NW = NC * NS
B_PER_W = N // NW
CHUNK = 320
NCHUNKS = B_PER_W // CHUNK


def _worker_id():
    return lax.axis_index("s") * NC + lax.axis_index("c")


@functools.cache
def _sc_kernels():
    mesh = plsc.VectorSubcoreMesh(core_axis_name="c", subcore_axis_name="s")

    @functools.partial(
        pl.kernel,
        out_type=jax.ShapeDtypeStruct((N, 2 * D), jnp.float32),
        mesh=mesh,
        compiler_params=pltpu.CompilerParams(use_tc_tiling_on_sc=True),
        scratch_types=[
            pltpu.VMEM((B_PER_W,), jnp.int32),
            pltpu.VMEM((CHUNK, 2 * D), jnp.float32),
            pltpu.VMEM((CHUNK, 2 * D), jnp.float32),
            pltpu.SemaphoreType.DMA,
            pltpu.SemaphoreType.DMA,
        ],
    )
    def base_gather(idxh_hbm, table_hbm, out_hbm, idx_v, b0, b1, sem0, sem1):
        base = _worker_id() * B_PER_W
        pltpu.sync_copy(idxh_hbm.at[pl.ds(base, B_PER_W)], idx_v)
        bufs = (b0, b1)
        sems = (sem0, sem1)
        cps = [None, None]

        def start(k):
            j = k % 2
            cps[j] = pltpu.async_copy(
                table_hbm.at[idx_v.at[pl.ds(k * CHUNK, CHUNK)]], bufs[j], sems[j])

        start(0)
        for k in range(NCHUNKS):
            if k + 1 < NCHUNKS:
                start(k + 1)
            j = k % 2
            cps[j].wait()
            pltpu.sync_copy(bufs[j], out_hbm.at[pl.ds(base + k * CHUNK, CHUNK)])

    @functools.partial(
        pl.kernel,
        out_type=jax.ShapeDtypeStruct((N, R), jnp.float32),
        mesh=mesh,
        compiler_params=pltpu.CompilerParams(use_tc_tiling_on_sc=False),
        scratch_types=[
            pltpu.VMEM((B_PER_W,), jnp.int32),
            pltpu.VMEM((B_PER_W, R), jnp.float32),
            pltpu.SemaphoreType.DMA,
        ],
    )
    def lora_gather(idx_hbm, at_hbm, out_hbm, idx_v, rows_v, sem):
        base = _worker_id() * B_PER_W
        pltpu.sync_copy(idx_hbm.at[pl.ds(base, B_PER_W)], idx_v)
        pltpu.async_copy(at_hbm.at[idx_v], rows_v, sem).wait()
        pltpu.sync_copy(rows_v, out_hbm.at[pl.ds(base, B_PER_W)])

    return base_gather, lora_gather


_VB = 2048


def _transpose_body(a_ref, out_ref):
    out_ref[...] = a_ref[...].T


_transpose = pl.pallas_call(
    _transpose_body,
    grid=(pl.cdiv(V, _VB),),
    in_specs=[pl.BlockSpec((R, _VB), lambda i: (0, i))],
    out_specs=pl.BlockSpec((_VB, R), lambda i: (i, 0)),
    out_shape=jax.ShapeDtypeStruct((V, R), jnp.float32),
)

_BN = 2048


def _fuse_body(ar_ref, rows_ref, par_ref, b_ref, out_ref):
    rows = rows_ref[...]
    left = rows[:, :D]
    right = rows[:, D:]
    base = jnp.where(par_ref[...] > 0.5, right, left)
    out_ref[...] = base + jnp.dot(
        ar_ref[...], b_ref[...].T, preferred_element_type=jnp.float32
    )


_fuse = pl.pallas_call(
    _fuse_body,
    grid=(N // _BN,),
    in_specs=[
        pl.BlockSpec((_BN, R), lambda i: (i, 0)),
        pl.BlockSpec((_BN, 2 * D), lambda i: (i, 0)),
        pl.BlockSpec((_BN, 1), lambda i: (i, 0)),
        pl.BlockSpec((D, R), lambda i: (0, 0)),
    ],
    out_specs=pl.BlockSpec((_BN, D), lambda i: (i, 0)),
    out_shape=jax.ShapeDtypeStruct((N, D), jnp.float32),
)


def kernel(x, base_weight, lora_A, lora_B):
    Bsz, Ssz = x.shape
    idx = x.reshape(-1)
    idxh = idx >> 1
    par = (idx & 1).astype(jnp.float32).reshape(N, 1)
    table128 = base_weight.reshape(V // 2, 2 * D)
    at = _transpose(lora_A)
    rows128 = _sc_kernels()[0](idxh, table128)
    ar = _sc_kernels()[1](idx, at)
    out = _fuse(ar, rows128, par, lora_B)
    return out.reshape(Bsz, Ssz, D)
